# Initial kernel scaffold; baseline (speedup 1.0000x reference)
import jax
import jax.numpy as jnp
from jax.experimental import pallas as pl


def kernel(pos, batch, params):
    raise NotImplementedError("write your pallas kernel here")


# speedup vs baseline: 6.2929x; 6.2929x over previous
import functools

import jax
import jax.numpy as jnp
from jax import lax
from jax.experimental import pallas as pl

_B = 32
_N0 = 1024
_INV_BN = float((1.0 + 1e-5) ** -0.5)


def _elu(v):
    return jnp.where(v > 0, v, jnp.exp(v) - 1.0)


def _row(a):
    return a.reshape(1, -1)


def _bcast_spec(shape):
    return pl.BlockSpec(shape, lambda *_: (0,) * len(shape))


def _prep_xconv_params(p, K, dm):
    Cm_dm = p['wo'].shape[0]
    Cm = Cm_dm // dm
    eye = jnp.eye(K, dtype=jnp.float32)
    wbd1 = jnp.einsum('ge,gtj->gtej', eye, jnp.transpose(p['wc1'], (0, 2, 1))
                      ).reshape(K * K, K * K)
    wbd2 = jnp.einsum('ge,gtj->gtej', eye, jnp.transpose(p['wc2'], (0, 2, 1))
                      ).reshape(K * K, K * K)
    wdt = jnp.transpose(p['wd'], (2, 1, 0)).reshape(K, dm * Cm)
    wop = p['wo'].reshape(Cm, dm, -1).transpose(1, 0, 2).reshape(dm * Cm, -1)
    bdp = p['bd'].reshape(Cm, dm).T.reshape(1, dm * Cm)
    return (
        p['w1'], _row(p['b1']), _row(p['g1'] * _INV_BN), _row(p['be1']),
        p['w2'], _row(p['b2']), _row(p['g2'] * _INV_BN), _row(p['be2']),
        p['wl'], _row(p['bl']), _row(p['gl'] * _INV_BN), _row(p['bel']),
        wbd1, _row(p['bc1']), _row(p['gc1'] * _INV_BN), _row(p['bec1']),
        wbd2, _row(p['bc2']), _row(p['gc2'] * _INV_BN), _row(p['bec2']),
        wdt, bdp, wop, _row(p['bo']),
    )


def _xconv_body(n, Tn, Cin, Cd, K, dil, dm, *refs):
    Kd = K * dil
    Cm = Cin + Cd
    if Cin:
        pos_ref, x_ref = refs[0], refs[1]
        prm = refs[2:26]
        out_ref = refs[26]
    else:
        pos_ref = refs[0]
        prm = refs[1:25]
        out_ref = refs[25]
    (w1, b1, g1, be1, w2, b2, g2, be2, wl, bl, gl, bel,
     wbd1, bc1, gc1, bec1, wbd2, bc2, gc2, bec2,
     wdt, bdp, wop, bop) = [r[...] for r in prm]

    posb = pos_ref[0]
    r0 = pl.program_id(1) * Tn
    pos_t = pos_ref[0, pl.ds(r0, Tn), :]
    if Cin:
        feat = jnp.concatenate([posb, x_ref[0]], axis=1)
    else:
        feat = posb

    gram = lax.dot_general(pos_t, posb, (((1,), (1,)), ((), ())))
    sq = jnp.sum(posb * posb, axis=1, keepdims=True)
    sq_t = jnp.sum(pos_t * pos_t, axis=1, keepdims=True)
    d2 = sq_t + jnp.transpose(sq) - 2.0 * gram

    iota = lax.broadcasted_iota(jnp.int32, (Tn, n), 1)
    rels = []
    xjs = []
    for k in range(Kd):
        vals = jnp.min(d2, axis=1, keepdims=True)
        cand = jnp.where(d2 == vals, iota, n)
        idx = jnp.min(cand, axis=1, keepdims=True)
        sel = iota == idx
        if k % dil == 0:
            g = lax.dot_general(sel.astype(jnp.float32), feat,
                                (((1,), (0,)), ((), ())))
            rels.append(g[:, :3] - pos_t)
            if Cin:
                xjs.append(g[:, 3:])
        if k < Kd - 1:
            d2 = jnp.where(sel, 1e30, d2)

    R = jnp.concatenate(rels, axis=0)
    h = _elu(R @ w1 + b1) * g1 + be1
    h = _elu(h @ w2 + b2) * g2 + be2

    RL = jnp.concatenate(rels, axis=1)
    t = _elu(RL @ wl + bl) * gl + bel
    t = _elu(t @ wbd1 + bc1) * gc1 + bec1
    t = (t @ wbd2 + bc2) * gc2 + bec2

    hk = []
    for k in range(K):
        hs = h[k * Tn:(k + 1) * Tn]
        hk.append(jnp.concatenate([hs, xjs[k]], axis=1) if Cin else hs)

    o = jnp.zeros((Tn, dm * Cm), jnp.float32)
    for j in range(K):
        acc = None
        for k in range(K):
            term = hk[k] * t[:, k * K + j:k * K + j + 1]
            acc = term if acc is None else acc + term
        htj = acc
        if dm > 1:
            htj = jnp.concatenate([htj] * dm, axis=1)
        o = o + htj * wdt[j:j + 1, :]
    o = o + bdp
    out_ref[0] = jnp.maximum(o @ wop + bop, 0.0)


def _xconv(pos3, x, prm, n, Cin, Cd, Cout, K, dil, dm, Tn=None):
    Tn = Tn or n
    body = functools.partial(_xconv_body, n, Tn, Cin, Cd, K, dil, dm)
    ins = [pos3] + ([x] if Cin else []) + list(prm)
    in_specs = [pl.BlockSpec((1, n, 3), lambda b, i: (b, 0, 0))]
    if Cin:
        in_specs.append(pl.BlockSpec((1, n, Cin), lambda b, i: (b, 0, 0)))
    in_specs += [_bcast_spec(a.shape) for a in prm]
    return pl.pallas_call(
        body,
        grid=(_B, n // Tn),
        in_specs=in_specs,
        out_specs=pl.BlockSpec((1, Tn, Cout), lambda b, i: (b, i, 0)),
        out_shape=jax.ShapeDtypeStruct((_B, n, Cout), jnp.float32),
    )(*ins)


def _fps_idx_body(n, m, pos_ref, idx_ref):
    pos = pos_ref[...]
    iota_n = lax.broadcasted_iota(jnp.int32, (_B, n), 1)
    iota_m = lax.broadcasted_iota(jnp.int32, (1, m), 1)

    def body(i, carry):
        center, mind, idxmat = carry
        diff = pos - center[:, :, None]
        d = jnp.sum(diff * diff, axis=1)
        mind = jnp.minimum(mind, d)
        vals = jnp.max(mind, axis=1, keepdims=True)
        cand = jnp.where(mind == vals, iota_n, n)
        nxt = jnp.min(cand, axis=1, keepdims=True)
        sel = (iota_n == nxt).astype(jnp.float32)
        center = jnp.sum(pos * sel[:, None, :], axis=2)
        idxmat = idxmat + nxt * (iota_m == i).astype(jnp.int32)
        return center, mind, idxmat

    center0 = pos[:, :, 0]
    mind0 = jnp.full((_B, n), 1e30, jnp.float32)
    idx0 = jnp.zeros((_B, m), jnp.int32)
    _, _, idxmat = lax.fori_loop(1, m, body, (center0, mind0, idx0))
    idx_ref[...] = idxmat


def _fps_gather_body(n, m, C, idx_ref, pos_ref, x_ref, pos_out_ref, x_out_ref):
    idxrow = idx_ref[0]
    featb = jnp.concatenate([pos_ref[0], x_ref[0]], axis=1)
    iota_nm = lax.broadcasted_iota(jnp.int32, (n, m), 0)
    ohT = (iota_nm == idxrow).astype(jnp.float32)
    g = lax.dot_general(ohT, featb, (((0,), (0,)), ((), ())))
    pos_out_ref[0] = g[:, :3]
    x_out_ref[0] = g[:, 3:]


def _fps(pos3, x, m):
    n, C = pos3.shape[1], x.shape[2]
    idx = pl.pallas_call(
        functools.partial(_fps_idx_body, n, m),
        out_shape=jax.ShapeDtypeStruct((_B, m), jnp.int32),
    )(jnp.transpose(pos3, (0, 2, 1)))
    idx3 = idx.reshape(_B, 1, m)
    return pl.pallas_call(
        functools.partial(_fps_gather_body, n, m, C),
        grid=(_B,),
        in_specs=[pl.BlockSpec((1, 1, m), lambda b: (b, 0, 0)),
                  pl.BlockSpec((1, n, 3), lambda b: (b, 0, 0)),
                  pl.BlockSpec((1, n, C), lambda b: (b, 0, 0))],
        out_specs=(pl.BlockSpec((1, m, 3), lambda b: (b, 0, 0)),
                   pl.BlockSpec((1, m, C), lambda b: (b, 0, 0))),
        out_shape=(jax.ShapeDtypeStruct((_B, m, 3), jnp.float32),
                   jax.ShapeDtypeStruct((_B, m, C), jnp.float32)),
    )(idx3, pos3, x)


def _head_body(n, x_ref, w1, b1, w2, b2, w3, b3, out_ref):
    x = jnp.sum(x_ref[...], axis=1) / float(n)
    x = jnp.maximum(x @ w1[...] + b1[...], 0.0)
    x = jnp.maximum(x @ w2[...] + b2[...], 0.0)
    x = x @ w3[...] + b3[...]
    mx = jnp.max(x, axis=1, keepdims=True)
    sh = x - mx
    out_ref[...] = sh - jnp.log(jnp.sum(jnp.exp(sh), axis=1, keepdims=True))


def _head(x4, lin1, lin2, lin3):
    n = x4.shape[1]
    nc = lin3['w'].shape[1]
    body = functools.partial(_head_body, n)
    return pl.pallas_call(
        body,
        out_shape=jax.ShapeDtypeStruct((_B, nc), jnp.float32),
    )(x4, lin1['w'], _row(lin1['b']), lin2['w'], _row(lin2['b']),
      lin3['w'], _row(lin3['b']))


def kernel(pos, batch, params):
    del batch
    pos3 = pos.reshape(_B, _N0, 3)
    c1 = _prep_xconv_params(params['conv1'], 8, 2)
    c2 = _prep_xconv_params(params['conv2'], 12, 1)
    c3 = _prep_xconv_params(params['conv3'], 16, 1)
    c4 = _prep_xconv_params(params['conv4'], 16, 1)

    x1 = _xconv(pos3, None, c1, _N0, 0, 32, 48, 8, 1, 2, Tn=256)
    pos2, x1s = _fps(pos3, x1, 384)
    x2 = _xconv(pos2, x1s, c2, 384, 48, 64, 96, 12, 2, 1)
    pos3b, x2s = _fps(pos2, x2, 129)
    x3 = _xconv(pos3b, x2s, c3, 129, 96, 128, 192, 16, 2, 1)
    x4 = _xconv(pos3b, x3, c4, 129, 192, 256, 384, 16, 2, 1)
    return _head(x4, params['lin1'], params['lin2'], params['lin3'])


# speedup vs baseline: 10.4835x; 1.6659x over previous
import functools

import jax
import jax.numpy as jnp
from jax import lax
from jax.experimental import pallas as pl

_B = 32
_N0 = 1024
_INV_BN = float((1.0 + 1e-5) ** -0.5)


def _elu(v):
    return jnp.where(v > 0, v, jnp.exp(v) - 1.0)


def _row(a):
    return a.reshape(1, -1)


def _bcast_spec(shape):
    return pl.BlockSpec(shape, lambda *_: (0,) * len(shape))


def _prep_xconv_params(p, K, dm):
    Cm_dm = p['wo'].shape[0]
    Cm = Cm_dm // dm
    eye = jnp.eye(K, dtype=jnp.float32)
    wbd1 = jnp.einsum('ge,gtj->gtej', eye, jnp.transpose(p['wc1'], (0, 2, 1))
                      ).reshape(K * K, K * K)
    wbd2 = jnp.einsum('ge,gtj->gtej', eye, jnp.transpose(p['wc2'], (0, 2, 1))
                      ).reshape(K * K, K * K)
    wdk = jnp.transpose(p['wd'], (1, 2, 0)).reshape(dm * K, Cm)
    wop = p['wo'].reshape(Cm, dm, -1).transpose(1, 0, 2).reshape(dm * Cm, -1)
    bdp = p['bd'].reshape(Cm, dm).T.reshape(1, dm * Cm)
    return (
        p['w1'], _row(p['b1']), _row(p['g1'] * _INV_BN), _row(p['be1']),
        p['w2'], _row(p['b2']), _row(p['g2'] * _INV_BN), _row(p['be2']),
        p['wl'], _row(p['bl']), _row(p['gl'] * _INV_BN), _row(p['bel']),
        wbd1, _row(p['bc1']), _row(p['gc1'] * _INV_BN), _row(p['bec1']),
        wbd2, _row(p['bc2']), _row(p['gc2'] * _INV_BN), _row(p['bec2']),
        wdk, bdp, wop, _row(p['bo']),
    )


def _xconv_body(n, Tn, Cin, Cd, K, dil, dm, *refs):
    Kd = K * dil
    Cm = Cin + Cd
    if Cin:
        pos_ref, x_ref = refs[0], refs[1]
        prm = refs[2:26]
        out_ref = refs[26]
    else:
        pos_ref = refs[0]
        prm = refs[1:25]
        out_ref = refs[25]
    (w1, b1, g1, be1, w2, b2, g2, be2, wl, bl, gl, bel,
     wbd1, bc1, gc1, bec1, wbd2, bc2, gc2, bec2,
     wdk, bdp, wop, bop) = [r[...] for r in prm]

    posb = pos_ref[0]
    r0 = pl.program_id(1) * Tn
    pos_t = pos_ref[0, pl.ds(r0, Tn), :]
    if Cin:
        feat = jnp.concatenate([posb, x_ref[0]], axis=1)
    else:
        feat = posb

    gram = lax.dot_general(pos_t, posb, (((1,), (1,)), ((), ())))
    sq = jnp.sum(posb * posb, axis=1, keepdims=True)
    sq_t = jnp.sum(pos_t * pos_t, axis=1, keepdims=True)
    d2 = sq_t + jnp.transpose(sq) - 2.0 * gram

    iota = lax.broadcasted_iota(jnp.int32, (Tn, n), 1)
    rels = []
    xjs = []
    for k in range(Kd):
        vals = jnp.min(d2, axis=1, keepdims=True)
        cand = jnp.where(d2 == vals, iota, n)
        idx = jnp.min(cand, axis=1, keepdims=True)
        sel = iota == idx
        if k % dil == 0:
            g = lax.dot_general(sel.astype(jnp.float32), feat,
                                (((1,), (0,)), ((), ())))
            rels.append(g[:, :3] - pos_t)
            if Cin:
                xjs.append(g[:, 3:])
        if k < Kd - 1:
            d2 = jnp.where(sel, 1e30, d2)

    R = jnp.concatenate(rels, axis=0)
    h = _elu(R @ w1 + b1) * g1 + be1
    h = _elu(h @ w2 + b2) * g2 + be2

    RL = jnp.concatenate(rels, axis=1)
    t = _elu(RL @ wl + bl) * gl + bel
    t = _elu(t @ wbd1 + bc1) * gc1 + bec1
    t = (t @ wbd2 + bc2) * gc2 + bec2

    hk = []
    for k in range(K):
        hs = h[k * Tn:(k + 1) * Tn]
        hk.append(jnp.concatenate([hs, xjs[k]], axis=1) if Cin else hs)

    o_parts = []
    for d in range(dm):
        wdk_d = wdk[d * K:(d + 1) * K, :]
        acc = None
        for k in range(K):
            s = t[:, k * K:(k + 1) * K] @ wdk_d
            term = hk[k] * s
            acc = term if acc is None else acc + term
        o_parts.append(acc)
    o = o_parts[0] if dm == 1 else jnp.concatenate(o_parts, axis=1)
    o = o + bdp
    out_ref[0] = jnp.maximum(o @ wop + bop, 0.0)


def _xconv(pos3, x, prm, n, Cin, Cd, Cout, K, dil, dm, Tn=None):
    Tn = Tn or n
    body = functools.partial(_xconv_body, n, Tn, Cin, Cd, K, dil, dm)
    ins = [pos3] + ([x] if Cin else []) + list(prm)
    in_specs = [pl.BlockSpec((1, n, 3), lambda b, i: (b, 0, 0))]
    if Cin:
        in_specs.append(pl.BlockSpec((1, n, Cin), lambda b, i: (b, 0, 0)))
    in_specs += [_bcast_spec(a.shape) for a in prm]
    return pl.pallas_call(
        body,
        grid=(_B, n // Tn),
        in_specs=in_specs,
        out_specs=pl.BlockSpec((1, Tn, Cout), lambda b, i: (b, i, 0)),
        out_shape=jax.ShapeDtypeStruct((_B, n, Cout), jnp.float32),
    )(*ins)


def _fps_idx_body(n, m, pos_ref, idx_ref):
    pos = pos_ref[...]
    iota_n = lax.broadcasted_iota(jnp.int32, (_B, n), 1)
    iota_m = lax.broadcasted_iota(jnp.int32, (1, m), 1)

    def body(i, carry):
        center, mind, idxmat = carry
        diff = pos - center[:, :, None]
        d = jnp.sum(diff * diff, axis=1)
        mind = jnp.minimum(mind, d)
        vals = jnp.max(mind, axis=1, keepdims=True)
        cand = jnp.where(mind == vals, iota_n, n)
        nxt = jnp.min(cand, axis=1, keepdims=True)
        sel = (iota_n == nxt).astype(jnp.float32)
        center = jnp.sum(pos * sel[:, None, :], axis=2)
        idxmat = idxmat + nxt * (iota_m == i).astype(jnp.int32)
        return center, mind, idxmat

    center0 = pos[:, :, 0]
    mind0 = jnp.full((_B, n), 1e30, jnp.float32)
    idx0 = jnp.zeros((_B, m), jnp.int32)
    _, _, idxmat = lax.fori_loop(1, m, body, (center0, mind0, idx0))
    idx_ref[...] = idxmat


def _fps_gather_body(n, m, C, idx_ref, pos_ref, x_ref, pos_out_ref, x_out_ref):
    idxrow = idx_ref[0]
    featb = jnp.concatenate([pos_ref[0], x_ref[0]], axis=1)
    iota_nm = lax.broadcasted_iota(jnp.int32, (n, m), 0)
    ohT = (iota_nm == idxrow).astype(jnp.float32)
    g = lax.dot_general(ohT, featb, (((0,), (0,)), ((), ())))
    pos_out_ref[0] = g[:, :3]
    x_out_ref[0] = g[:, 3:]


def _fps(pos3, x, m):
    n, C = pos3.shape[1], x.shape[2]
    idx = pl.pallas_call(
        functools.partial(_fps_idx_body, n, m),
        out_shape=jax.ShapeDtypeStruct((_B, m), jnp.int32),
    )(jnp.transpose(pos3, (0, 2, 1)))
    idx3 = idx.reshape(_B, 1, m)
    return pl.pallas_call(
        functools.partial(_fps_gather_body, n, m, C),
        grid=(_B,),
        in_specs=[pl.BlockSpec((1, 1, m), lambda b: (b, 0, 0)),
                  pl.BlockSpec((1, n, 3), lambda b: (b, 0, 0)),
                  pl.BlockSpec((1, n, C), lambda b: (b, 0, 0))],
        out_specs=(pl.BlockSpec((1, m, 3), lambda b: (b, 0, 0)),
                   pl.BlockSpec((1, m, C), lambda b: (b, 0, 0))),
        out_shape=(jax.ShapeDtypeStruct((_B, m, 3), jnp.float32),
                   jax.ShapeDtypeStruct((_B, m, C), jnp.float32)),
    )(idx3, pos3, x)


def _head_body(n, x_ref, w1, b1, w2, b2, w3, b3, out_ref):
    x = jnp.sum(x_ref[...], axis=1) / float(n)
    x = jnp.maximum(x @ w1[...] + b1[...], 0.0)
    x = jnp.maximum(x @ w2[...] + b2[...], 0.0)
    x = x @ w3[...] + b3[...]
    mx = jnp.max(x, axis=1, keepdims=True)
    sh = x - mx
    out_ref[...] = sh - jnp.log(jnp.sum(jnp.exp(sh), axis=1, keepdims=True))


def _head(x4, lin1, lin2, lin3):
    n = x4.shape[1]
    nc = lin3['w'].shape[1]
    body = functools.partial(_head_body, n)
    return pl.pallas_call(
        body,
        out_shape=jax.ShapeDtypeStruct((_B, nc), jnp.float32),
    )(x4, lin1['w'], _row(lin1['b']), lin2['w'], _row(lin2['b']),
      lin3['w'], _row(lin3['b']))


def kernel(pos, batch, params):
    del batch
    pos3 = pos.reshape(_B, _N0, 3)
    c1 = _prep_xconv_params(params['conv1'], 8, 2)
    c2 = _prep_xconv_params(params['conv2'], 12, 1)
    c3 = _prep_xconv_params(params['conv3'], 16, 1)
    c4 = _prep_xconv_params(params['conv4'], 16, 1)

    x1 = _xconv(pos3, None, c1, _N0, 0, 32, 48, 8, 1, 2, Tn=256)
    pos2, x1s = _fps(pos3, x1, 384)
    x2 = _xconv(pos2, x1s, c2, 384, 48, 64, 96, 12, 2, 1)
    pos3b, x2s = _fps(pos2, x2, 129)
    x3 = _xconv(pos3b, x2s, c3, 129, 96, 128, 192, 16, 2, 1)
    x4 = _xconv(pos3b, x3, c4, 129, 192, 256, 384, 16, 2, 1)
    return _head(x4, params['lin1'], params['lin2'], params['lin3'])


# speedup vs baseline: 12.1070x; 1.1549x over previous
import functools

import jax
import jax.numpy as jnp
from jax import lax
from jax.experimental import pallas as pl

_B = 32
_N0 = 1024
_INV_BN = float((1.0 + 1e-5) ** -0.5)


def _elu(v):
    return jnp.where(v > 0, v, jnp.exp(v) - 1.0)


def _row(a):
    return a.reshape(1, -1)


def _bcast_spec(shape):
    return pl.BlockSpec(shape, lambda *_: (0,) * len(shape))


def _prep_xconv_params(p, K, dm):
    Cm_dm = p['wo'].shape[0]
    Cm = Cm_dm // dm
    eye = jnp.eye(K, dtype=jnp.float32)
    wbd1 = jnp.einsum('ge,gtj->gtej', eye, jnp.transpose(p['wc1'], (0, 2, 1))
                      ).reshape(K * K, K * K)
    wbd2 = jnp.einsum('ge,gtj->gtej', eye, jnp.transpose(p['wc2'], (0, 2, 1))
                      ).reshape(K * K, K * K)
    wdk = jnp.transpose(p['wd'], (1, 2, 0)).reshape(dm * K, Cm)
    wop = p['wo'].reshape(Cm, dm, -1).transpose(1, 0, 2).reshape(dm * Cm, -1)
    bdp = p['bd'].reshape(Cm, dm).T.reshape(1, dm * Cm)
    return (
        p['w1'], _row(p['b1']), _row(p['g1'] * _INV_BN), _row(p['be1']),
        p['w2'], _row(p['b2']), _row(p['g2'] * _INV_BN), _row(p['be2']),
        p['wl'], _row(p['bl']), _row(p['gl'] * _INV_BN), _row(p['bel']),
        wbd1, _row(p['bc1']), _row(p['gc1'] * _INV_BN), _row(p['bec1']),
        wbd2, _row(p['bc2']), _row(p['gc2'] * _INV_BN), _row(p['bec2']),
        wdk, bdp, wop, _row(p['bo']),
    )


def _xconv_body(n, Tn, Cin, Cd, K, dil, dm, *refs):
    Kd = K * dil
    Cm = Cin + Cd
    if Cin:
        pos_ref, x_ref = refs[0], refs[1]
        prm = refs[2:26]
        out_ref = refs[26]
    else:
        pos_ref = refs[0]
        prm = refs[1:25]
        out_ref = refs[25]
    (w1, b1, g1, be1, w2, b2, g2, be2, wl, bl, gl, bel,
     wbd1, bc1, gc1, bec1, wbd2, bc2, gc2, bec2,
     wdk, bdp, wop, bop) = [r[...] for r in prm]

    posb = pos_ref[0]
    r0 = pl.program_id(1) * Tn
    pos_t = pos_ref[0, pl.ds(r0, Tn), :]
    if Cin:
        feat = jnp.concatenate([posb, x_ref[0]], axis=1)
    else:
        feat = posb

    gram = lax.dot_general(pos_t, posb, (((1,), (1,)), ((), ())))
    sq = jnp.sum(posb * posb, axis=1, keepdims=True)
    sq_t = jnp.sum(pos_t * pos_t, axis=1, keepdims=True)
    d2 = sq_t + jnp.transpose(sq) - 2.0 * gram

    iota = lax.broadcasted_iota(jnp.int32, (Tn, n), 1).astype(jnp.float32)
    rels = []
    xjs = []
    for k in range(Kd):
        vals = jnp.min(d2, axis=1, keepdims=True)
        cand = jnp.where(d2 == vals, iota, float(n))
        idx = jnp.min(cand, axis=1, keepdims=True)
        sel = iota == idx
        if k % dil == 0:
            g = lax.dot_general(sel.astype(jnp.float32), feat,
                                (((1,), (0,)), ((), ())))
            rels.append(g[:, :3] - pos_t)
            if Cin:
                xjs.append(g[:, 3:])
        if k < Kd - 1:
            d2 = jnp.where(sel, 1e30, d2)

    R = jnp.concatenate(rels, axis=0)
    h = _elu(R @ w1 + b1) * g1 + be1
    h = _elu(h @ w2 + b2) * g2 + be2

    RL = jnp.concatenate(rels, axis=1)
    t = _elu(RL @ wl + bl) * gl + bel
    t = _elu(t @ wbd1 + bc1) * gc1 + bec1
    t = (t @ wbd2 + bc2) * gc2 + bec2

    hk = []
    for k in range(K):
        hs = h[k * Tn:(k + 1) * Tn]
        hk.append(jnp.concatenate([hs, xjs[k]], axis=1) if Cin else hs)

    o_parts = []
    for d in range(dm):
        wdk_d = wdk[d * K:(d + 1) * K, :]
        acc = None
        for k in range(K):
            s = t[:, k * K:(k + 1) * K] @ wdk_d
            term = hk[k] * s
            acc = term if acc is None else acc + term
        o_parts.append(acc)
    o = o_parts[0] if dm == 1 else jnp.concatenate(o_parts, axis=1)
    o = o + bdp
    out_ref[0] = jnp.maximum(o @ wop + bop, 0.0)


def _xconv(pos3, x, prm, n, Cin, Cd, Cout, K, dil, dm, Tn=None):
    Tn = Tn or n
    body = functools.partial(_xconv_body, n, Tn, Cin, Cd, K, dil, dm)
    ins = [pos3] + ([x] if Cin else []) + list(prm)
    in_specs = [pl.BlockSpec((1, n, 3), lambda b, i: (b, 0, 0))]
    if Cin:
        in_specs.append(pl.BlockSpec((1, n, Cin), lambda b, i: (b, 0, 0)))
    in_specs += [_bcast_spec(a.shape) for a in prm]
    return pl.pallas_call(
        body,
        grid=(_B, n // Tn),
        in_specs=in_specs,
        out_specs=pl.BlockSpec((1, Tn, Cout), lambda b, i: (b, i, 0)),
        out_shape=jax.ShapeDtypeStruct((_B, n, Cout), jnp.float32),
    )(*ins)


def _fps_idx_body(n, m, pos_ref, idx_ref):
    pos = pos_ref[...]
    iota_n = lax.broadcasted_iota(jnp.int32, (_B, n), 1).astype(jnp.float32)
    iota_m = lax.broadcasted_iota(jnp.int32, (1, m), 1)

    def body(i, carry):
        center, mind, idxmat = carry
        diff = pos - center[:, :, None]
        d = jnp.sum(diff * diff, axis=1)
        mind = jnp.minimum(mind, d)
        vals = jnp.max(mind, axis=1, keepdims=True)
        cand = jnp.where(mind == vals, iota_n, float(n))
        nxt = jnp.min(cand, axis=1, keepdims=True)
        sel = (iota_n == nxt).astype(jnp.float32)
        center = jnp.sum(pos * sel[:, None, :], axis=2)
        idxmat = idxmat + nxt.astype(jnp.int32) * (iota_m == i).astype(jnp.int32)
        return center, mind, idxmat

    center0 = pos[:, :, 0]
    mind0 = jnp.full((_B, n), 1e30, jnp.float32)
    idx0 = jnp.zeros((_B, m), jnp.int32)
    _, _, idxmat = lax.fori_loop(1, m, body, (center0, mind0, idx0))
    idx_ref[...] = idxmat


def _fps_gather_body(n, m, C, idx_ref, pos_ref, x_ref, pos_out_ref, x_out_ref):
    idxrow = idx_ref[0]
    featb = jnp.concatenate([pos_ref[0], x_ref[0]], axis=1)
    iota_nm = lax.broadcasted_iota(jnp.int32, (n, m), 0)
    ohT = (iota_nm == idxrow).astype(jnp.float32)
    g = lax.dot_general(ohT, featb, (((0,), (0,)), ((), ())))
    pos_out_ref[0] = g[:, :3]
    x_out_ref[0] = g[:, 3:]


def _fps(pos3, x, m):
    n, C = pos3.shape[1], x.shape[2]
    idx = pl.pallas_call(
        functools.partial(_fps_idx_body, n, m),
        out_shape=jax.ShapeDtypeStruct((_B, m), jnp.int32),
    )(jnp.transpose(pos3, (0, 2, 1)))
    idx3 = idx.reshape(_B, 1, m)
    return pl.pallas_call(
        functools.partial(_fps_gather_body, n, m, C),
        grid=(_B,),
        in_specs=[pl.BlockSpec((1, 1, m), lambda b: (b, 0, 0)),
                  pl.BlockSpec((1, n, 3), lambda b: (b, 0, 0)),
                  pl.BlockSpec((1, n, C), lambda b: (b, 0, 0))],
        out_specs=(pl.BlockSpec((1, m, 3), lambda b: (b, 0, 0)),
                   pl.BlockSpec((1, m, C), lambda b: (b, 0, 0))),
        out_shape=(jax.ShapeDtypeStruct((_B, m, 3), jnp.float32),
                   jax.ShapeDtypeStruct((_B, m, C), jnp.float32)),
    )(idx3, pos3, x)


def _head_body(n, x_ref, w1, b1, w2, b2, w3, b3, out_ref):
    x = jnp.sum(x_ref[...], axis=1) / float(n)
    x = jnp.maximum(x @ w1[...] + b1[...], 0.0)
    x = jnp.maximum(x @ w2[...] + b2[...], 0.0)
    x = x @ w3[...] + b3[...]
    mx = jnp.max(x, axis=1, keepdims=True)
    sh = x - mx
    out_ref[...] = sh - jnp.log(jnp.sum(jnp.exp(sh), axis=1, keepdims=True))


def _head(x4, lin1, lin2, lin3):
    n = x4.shape[1]
    nc = lin3['w'].shape[1]
    body = functools.partial(_head_body, n)
    return pl.pallas_call(
        body,
        out_shape=jax.ShapeDtypeStruct((_B, nc), jnp.float32),
    )(x4, lin1['w'], _row(lin1['b']), lin2['w'], _row(lin2['b']),
      lin3['w'], _row(lin3['b']))


def kernel(pos, batch, params):
    del batch
    pos3 = pos.reshape(_B, _N0, 3)
    c1 = _prep_xconv_params(params['conv1'], 8, 2)
    c2 = _prep_xconv_params(params['conv2'], 12, 1)
    c3 = _prep_xconv_params(params['conv3'], 16, 1)
    c4 = _prep_xconv_params(params['conv4'], 16, 1)

    x1 = _xconv(pos3, None, c1, _N0, 0, 32, 48, 8, 1, 2, Tn=256)
    pos2, x1s = _fps(pos3, x1, 384)
    x2 = _xconv(pos2, x1s, c2, 384, 48, 64, 96, 12, 2, 1)
    pos3b, x2s = _fps(pos2, x2, 129)
    x3 = _xconv(pos3b, x2s, c3, 129, 96, 128, 192, 16, 2, 1)
    x4 = _xconv(pos3b, x3, c4, 129, 192, 256, 384, 16, 2, 1)
    return _head(x4, params['lin1'], params['lin2'], params['lin3'])


# speedup vs baseline: 13.1159x; 1.0833x over previous
import functools

import jax
import jax.numpy as jnp
from jax import lax
from jax.experimental import pallas as pl

_B = 32
_N0 = 1024
_INV_BN = float((1.0 + 1e-5) ** -0.5)


def _elu(v):
    return jnp.where(v > 0, v, jnp.exp(v) - 1.0)


def _row(a):
    return a.reshape(1, -1)


def _bcast_spec(shape):
    return pl.BlockSpec(shape, lambda *_: (0,) * len(shape))


def _prep_xconv_params(p, K, dm):
    Cm_dm = p['wo'].shape[0]
    Cm = Cm_dm // dm
    eye = jnp.eye(K, dtype=jnp.float32)
    wbd1 = jnp.einsum('ge,gtj->gtej', eye, jnp.transpose(p['wc1'], (0, 2, 1))
                      ).reshape(K * K, K * K)
    wbd2 = jnp.einsum('ge,gtj->gtej', eye, jnp.transpose(p['wc2'], (0, 2, 1))
                      ).reshape(K * K, K * K)
    wdk = jnp.transpose(p['wd'], (1, 2, 0)).reshape(dm * K, Cm)
    wop = p['wo'].reshape(Cm, dm, -1).transpose(1, 0, 2).reshape(dm * Cm, -1)
    bdp = p['bd'].reshape(Cm, dm).T.reshape(1, dm * Cm)
    return (
        p['w1'], _row(p['b1']), _row(p['g1'] * _INV_BN), _row(p['be1']),
        p['w2'], _row(p['b2']), _row(p['g2'] * _INV_BN), _row(p['be2']),
        p['wl'], _row(p['bl']), _row(p['gl'] * _INV_BN), _row(p['bel']),
        wbd1, _row(p['bc1']), _row(p['gc1'] * _INV_BN), _row(p['bec1']),
        wbd2, _row(p['bc2']), _row(p['gc2'] * _INV_BN), _row(p['bec2']),
        wdk, bdp, wop, _row(p['bo']),
    )


def _knn_sels(pos_t, posb, Kd, dil, n, Tn):
    gram = lax.dot_general(pos_t, posb, (((1,), (1,)), ((), ())))
    sq = jnp.sum(posb * posb, axis=1, keepdims=True)
    sq_t = jnp.sum(pos_t * pos_t, axis=1, keepdims=True)
    d2 = sq_t + jnp.transpose(sq) - 2.0 * gram
    iota = lax.broadcasted_iota(jnp.int32, (Tn, n), 1).astype(jnp.float32)
    sels = []
    for k in range(Kd):
        vals = jnp.min(d2, axis=1, keepdims=True)
        cand = jnp.where(d2 == vals, iota, float(n))
        idx = jnp.min(cand, axis=1, keepdims=True)
        sel = iota == idx
        if k % dil == 0:
            sels.append(sel.astype(jnp.float32))
        if k < Kd - 1:
            d2 = jnp.where(sel, 1e30, d2)
    return sels


def _xconv_dense(pos_t, feat, sels, prm, Cin, Cd, K, dm, Tn):
    (w1, b1, g1, be1, w2, b2, g2, be2, wl, bl, gl, bel,
     wbd1, bc1, gc1, bec1, wbd2, bc2, gc2, bec2,
     wdk, bdp, wop, bop) = prm
    Cm = Cin + Cd
    rels = []
    xjs = []
    for s in sels:
        g = lax.dot_general(s, feat, (((1,), (0,)), ((), ())))
        rels.append(g[:, :3] - pos_t)
        if Cin:
            xjs.append(g[:, 3:])

    R = jnp.concatenate(rels, axis=0)
    h = _elu(R @ w1 + b1) * g1 + be1
    h = _elu(h @ w2 + b2) * g2 + be2

    RL = jnp.concatenate(rels, axis=1)
    t = _elu(RL @ wl + bl) * gl + bel
    t = _elu(t @ wbd1 + bc1) * gc1 + bec1
    t = (t @ wbd2 + bc2) * gc2 + bec2

    hk = []
    for k in range(K):
        hs = h[k * Tn:(k + 1) * Tn]
        hk.append(jnp.concatenate([hs, xjs[k]], axis=1) if Cin else hs)

    o_parts = []
    for d in range(dm):
        wdk_d = wdk[d * K:(d + 1) * K, :]
        acc = None
        for k in range(K):
            s = t[:, k * K:(k + 1) * K] @ wdk_d
            term = hk[k] * s
            acc = term if acc is None else acc + term
        o_parts.append(acc)
    o = o_parts[0] if dm == 1 else jnp.concatenate(o_parts, axis=1)
    o = o + bdp
    return jnp.maximum(o @ wop + bop, 0.0)


def _xconv_body(n, Tn, Cin, Cd, K, dil, dm, *refs):
    if Cin:
        pos_ref, x_ref = refs[0], refs[1]
        prm = refs[2:26]
        out_ref = refs[26]
    else:
        pos_ref = refs[0]
        prm = refs[1:25]
        out_ref = refs[25]
    prm = [r[...] for r in prm]
    posb = pos_ref[0]
    r0 = pl.program_id(1) * Tn
    pos_t = pos_ref[0, pl.ds(r0, Tn), :]
    feat = jnp.concatenate([posb, x_ref[0]], axis=1) if Cin else posb
    sels = _knn_sels(pos_t, posb, K * dil, dil, n, Tn)
    out_ref[0] = _xconv_dense(pos_t, feat, sels, prm, Cin, Cd, K, dm, Tn)


def _xconv34_body(n, Cin3, Cd3, Cd4, K, dil, *refs):
    pos_ref, x_ref = refs[0], refs[1]
    prm3 = [r[...] for r in refs[2:26]]
    prm4 = [r[...] for r in refs[26:50]]
    out_ref = refs[50]
    posb = pos_ref[0]
    feat3 = jnp.concatenate([posb, x_ref[0]], axis=1)
    sels = _knn_sels(posb, posb, K * dil, dil, n, n)
    x3 = _xconv_dense(posb, feat3, sels, prm3, Cin3, Cd3, K, 1, n)
    feat4 = jnp.concatenate([posb, x3], axis=1)
    x4 = _xconv_dense(posb, feat4, sels, prm4, x3.shape[1], Cd4, K, 1, n)
    out_ref[0] = jnp.sum(x4, axis=0, keepdims=True) / float(n)


def _xconv(pos3, x, prm, n, Cin, Cd, Cout, K, dil, dm, Tn=None):
    Tn = Tn or n
    body = functools.partial(_xconv_body, n, Tn, Cin, Cd, K, dil, dm)
    ins = [pos3] + ([x] if Cin else []) + list(prm)
    in_specs = [pl.BlockSpec((1, n, 3), lambda b, i: (b, 0, 0))]
    if Cin:
        in_specs.append(pl.BlockSpec((1, n, Cin), lambda b, i: (b, 0, 0)))
    in_specs += [_bcast_spec(a.shape) for a in prm]
    return pl.pallas_call(
        body,
        grid=(_B, n // Tn),
        in_specs=in_specs,
        out_specs=pl.BlockSpec((1, Tn, Cout), lambda b, i: (b, i, 0)),
        out_shape=jax.ShapeDtypeStruct((_B, n, Cout), jnp.float32),
    )(*ins)


def _fps_idx_body(n, m, pos_ref, idx_ref):
    pos = pos_ref[...]
    iota_n = lax.broadcasted_iota(jnp.int32, (_B, n), 1).astype(jnp.float32)
    iota_m = lax.broadcasted_iota(jnp.int32, (1, m), 1)

    def body(i, carry):
        center, mind, idxmat = carry
        diff = pos - center[:, :, None]
        d = jnp.sum(diff * diff, axis=1)
        mind = jnp.minimum(mind, d)
        vals = jnp.max(mind, axis=1, keepdims=True)
        cand = jnp.where(mind == vals, iota_n, float(n))
        nxt = jnp.min(cand, axis=1, keepdims=True)
        sel = (iota_n == nxt).astype(jnp.float32)
        center = jnp.sum(pos * sel[:, None, :], axis=2)
        idxmat = idxmat + nxt.astype(jnp.int32) * (iota_m == i).astype(jnp.int32)
        return center, mind, idxmat

    center0 = pos[:, :, 0]
    mind0 = jnp.full((_B, n), 1e30, jnp.float32)
    idx0 = jnp.zeros((_B, m), jnp.int32)
    _, _, idxmat = lax.fori_loop(1, m, body, (center0, mind0, idx0))
    idx_ref[...] = idxmat


def _fps_gather_body(n, m, C, idx_ref, pos_ref, x_ref, pos_out_ref, x_out_ref):
    idxrow = idx_ref[0]
    featb = jnp.concatenate([pos_ref[0], x_ref[0]], axis=1)
    iota_nm = lax.broadcasted_iota(jnp.int32, (n, m), 0)
    ohT = (iota_nm == idxrow).astype(jnp.float32)
    g = lax.dot_general(ohT, featb, (((0,), (0,)), ((), ())))
    pos_out_ref[0] = g[:, :3]
    x_out_ref[0] = g[:, 3:]


def _fps(pos3, x, m):
    n, C = pos3.shape[1], x.shape[2]
    idx = pl.pallas_call(
        functools.partial(_fps_idx_body, n, m),
        out_shape=jax.ShapeDtypeStruct((_B, m), jnp.int32),
    )(jnp.transpose(pos3, (0, 2, 1)))
    idx3 = idx.reshape(_B, 1, m)
    return pl.pallas_call(
        functools.partial(_fps_gather_body, n, m, C),
        grid=(_B,),
        in_specs=[pl.BlockSpec((1, 1, m), lambda b: (b, 0, 0)),
                  pl.BlockSpec((1, n, 3), lambda b: (b, 0, 0)),
                  pl.BlockSpec((1, n, C), lambda b: (b, 0, 0))],
        out_specs=(pl.BlockSpec((1, m, 3), lambda b: (b, 0, 0)),
                   pl.BlockSpec((1, m, C), lambda b: (b, 0, 0))),
        out_shape=(jax.ShapeDtypeStruct((_B, m, 3), jnp.float32),
                   jax.ShapeDtypeStruct((_B, m, C), jnp.float32)),
    )(idx3, pos3, x)


def _xconv34(pos3, x, prm3, prm4, n, Cin3, Cd3, Cd4, Cout4, K, dil):
    body = functools.partial(_xconv34_body, n, Cin3, Cd3, Cd4, K, dil)
    ins = [pos3, x] + list(prm3) + list(prm4)
    in_specs = [pl.BlockSpec((1, n, 3), lambda b: (b, 0, 0)),
                pl.BlockSpec((1, n, Cin3), lambda b: (b, 0, 0))]
    in_specs += [_bcast_spec(a.shape) for a in list(prm3) + list(prm4)]
    out = pl.pallas_call(
        body,
        grid=(_B,),
        in_specs=in_specs,
        out_specs=pl.BlockSpec((1, 1, Cout4), lambda b: (b, 0, 0)),
        out_shape=jax.ShapeDtypeStruct((_B, 1, Cout4), jnp.float32),
    )(*ins)
    return out.reshape(_B, Cout4)


def _head_body(x_ref, w1, b1, w2, b2, w3, b3, out_ref):
    x = x_ref[...]
    x = jnp.maximum(x @ w1[...] + b1[...], 0.0)
    x = jnp.maximum(x @ w2[...] + b2[...], 0.0)
    x = x @ w3[...] + b3[...]
    mx = jnp.max(x, axis=1, keepdims=True)
    sh = x - mx
    out_ref[...] = sh - jnp.log(jnp.sum(jnp.exp(sh), axis=1, keepdims=True))


def _head(xmean, lin1, lin2, lin3):
    nc = lin3['w'].shape[1]
    return pl.pallas_call(
        _head_body,
        out_shape=jax.ShapeDtypeStruct((_B, nc), jnp.float32),
    )(xmean, lin1['w'], _row(lin1['b']), lin2['w'], _row(lin2['b']),
      lin3['w'], _row(lin3['b']))


def kernel(pos, batch, params):
    del batch
    pos3 = pos.reshape(_B, _N0, 3)
    c1 = _prep_xconv_params(params['conv1'], 8, 2)
    c2 = _prep_xconv_params(params['conv2'], 12, 1)
    c3 = _prep_xconv_params(params['conv3'], 16, 1)
    c4 = _prep_xconv_params(params['conv4'], 16, 1)

    x1 = _xconv(pos3, None, c1, _N0, 0, 32, 48, 8, 1, 2, Tn=256)
    pos2, x1s = _fps(pos3, x1, 384)
    x2 = _xconv(pos2, x1s, c2, 384, 48, 64, 96, 12, 2, 1)
    pos3b, x2s = _fps(pos2, x2, 129)
    xmean = _xconv34(pos3b, x2s, c3, c4, 129, 96, 128, 256, 384, 16, 2)
    return _head(xmean, params['lin1'], params['lin2'], params['lin3'])


# speedup vs baseline: 13.8098x; 1.0529x over previous
import functools

import jax
import jax.numpy as jnp
from jax import lax
from jax.experimental import pallas as pl

_B = 32
_N0 = 1024
_INV_BN = float((1.0 + 1e-5) ** -0.5)


def _elu(v):
    return jnp.where(v > 0, v, jnp.exp(v) - 1.0)


def _row(a):
    return a.reshape(1, -1)


def _bcast_spec(shape):
    return pl.BlockSpec(shape, lambda *_: (0,) * len(shape))


def _prep_xconv_params(p, K, dm):
    Cm_dm = p['wo'].shape[0]
    Cm = Cm_dm // dm
    eye = jnp.eye(K, dtype=jnp.float32)
    wbd1 = jnp.einsum('ge,gtj->gtej', eye, jnp.transpose(p['wc1'], (0, 2, 1))
                      ).reshape(K * K, K * K)
    wbd2 = jnp.einsum('ge,gtj->gtej', eye, jnp.transpose(p['wc2'], (0, 2, 1))
                      ).reshape(K * K, K * K)
    wdk = jnp.transpose(p['wd'], (1, 2, 0)).reshape(dm * K, Cm)
    wop = p['wo'].reshape(Cm, dm, -1).transpose(1, 0, 2).reshape(dm * Cm, -1)
    bdp = p['bd'].reshape(Cm, dm).T.reshape(1, dm * Cm)
    return (
        p['w1'], _row(p['b1']), _row(p['g1'] * _INV_BN), _row(p['be1']),
        p['w2'], _row(p['b2']), _row(p['g2'] * _INV_BN), _row(p['be2']),
        p['wl'], _row(p['bl']), _row(p['gl'] * _INV_BN), _row(p['bel']),
        wbd1, _row(p['bc1']), _row(p['gc1'] * _INV_BN), _row(p['bec1']),
        wbd2, _row(p['bc2']), _row(p['gc2'] * _INV_BN), _row(p['bec2']),
        wdk, bdp, wop, _row(p['bo']),
    )


def _knn_sels(pos_t, posb, Kd, dil, n, Tn):
    gram = lax.dot_general(pos_t, posb, (((1,), (1,)), ((), ())))
    sq = jnp.sum(posb * posb, axis=1, keepdims=True)
    sq_t = jnp.sum(pos_t * pos_t, axis=1, keepdims=True)
    d2 = sq_t + jnp.transpose(sq) - 2.0 * gram
    iota = lax.broadcasted_iota(jnp.int32, (Tn, n), 1).astype(jnp.float32)
    sels = []
    for k in range(Kd):
        vals = jnp.min(d2, axis=1, keepdims=True)
        cand = jnp.where(d2 == vals, iota, float(n))
        idx = jnp.min(cand, axis=1, keepdims=True)
        sel = iota == idx
        if k % dil == 0:
            sels.append(sel.astype(jnp.float32))
        if k < Kd - 1:
            d2 = jnp.where(sel, 1e30, d2)
    return sels


def _xconv_dense(pos_t, feat, sels, prm, Cin, Cd, K, dm, Tn):
    (w1, b1, g1, be1, w2, b2, g2, be2, wl, bl, gl, bel,
     wbd1, bc1, gc1, bec1, wbd2, bc2, gc2, bec2,
     wdk, bdp, wop, bop) = prm
    Cm = Cin + Cd
    rels = []
    xjs = []
    for s in sels:
        g = lax.dot_general(s, feat, (((1,), (0,)), ((), ())))
        rels.append(g[:, :3] - pos_t)
        if Cin:
            xjs.append(g[:, 3:])

    R = jnp.concatenate(rels, axis=0)
    h = _elu(R @ w1 + b1) * g1 + be1
    h = _elu(h @ w2 + b2) * g2 + be2

    RL = jnp.concatenate(rels, axis=1)
    t = _elu(RL @ wl + bl) * gl + bel
    t = _elu(t @ wbd1 + bc1) * gc1 + bec1
    t = (t @ wbd2 + bc2) * gc2 + bec2

    hk = []
    for k in range(K):
        hs = h[k * Tn:(k + 1) * Tn]
        hk.append(jnp.concatenate([hs, xjs[k]], axis=1) if Cin else hs)

    o_parts = []
    for d in range(dm):
        wdk_d = wdk[d * K:(d + 1) * K, :]
        acc = None
        for k in range(K):
            s = t[:, k * K:(k + 1) * K] @ wdk_d
            term = hk[k] * s
            acc = term if acc is None else acc + term
        o_parts.append(acc)
    o = o_parts[0] if dm == 1 else jnp.concatenate(o_parts, axis=1)
    o = o + bdp
    return jnp.maximum(o @ wop + bop, 0.0)


def _xconv1_body(n, Tn, Cd, K, dil, dm, pos_ref, *refs):
    prm = [r[...] for r in refs[:24]]
    out_ref, post_ref = refs[24], refs[25]
    posb = pos_ref[0]
    r0 = pl.program_id(1) * Tn
    pos_t = pos_ref[0, pl.ds(r0, Tn), :]
    sels = _knn_sels(pos_t, posb, K * dil, dil, n, Tn)
    out_ref[0] = _xconv_dense(pos_t, posb, sels, prm, 0, Cd, K, dm, Tn)
    post_ref[0] = jnp.transpose(pos_t)


def _gather_feat(idxrow, pos_ref, x_ref, n, m):
    featb = jnp.concatenate([pos_ref[0], x_ref[0]], axis=1)
    iota_nm = lax.broadcasted_iota(jnp.int32, (n, m), 0)
    ohT = (iota_nm == idxrow).astype(jnp.float32)
    return lax.dot_general(ohT, featb, (((0,), (0,)), ((), ())))


def _xconv2_body(n, m, Cin, Cd, K, dil, dm, idx_ref, pos_ref, x_ref, *refs):
    prm = [r[...] for r in refs[:24]]
    out_ref, pos2_ref, pos2t_ref = refs[24], refs[25], refs[26]
    g = _gather_feat(idx_ref[0], pos_ref, x_ref, n, m)
    pos2 = g[:, :3]
    sels = _knn_sels(pos2, pos2, K * dil, dil, m, m)
    out_ref[0] = _xconv_dense(pos2, g, sels, prm, Cin, Cd, K, dm, m)
    pos2_ref[0] = pos2
    pos2t_ref[0] = jnp.transpose(pos2)


def _xconv34_body(n, m, Cin3, Cd3, Cd4, K, dil, *refs):
    idx_ref, pos_ref, x_ref = refs[0], refs[1], refs[2]
    prm3 = [r[...] for r in refs[3:27]]
    prm4 = [r[...] for r in refs[27:51]]
    out_ref = refs[51]
    g = _gather_feat(idx_ref[0], pos_ref, x_ref, n, m)
    posb = g[:, :3]
    sels = _knn_sels(posb, posb, K * dil, dil, m, m)
    x3 = _xconv_dense(posb, g, sels, prm3, Cin3, Cd3, K, 1, m)
    feat4 = jnp.concatenate([posb, x3], axis=1)
    x4 = _xconv_dense(posb, feat4, sels, prm4, x3.shape[1], Cd4, K, 1, m)
    out_ref[0] = jnp.sum(x4, axis=0, keepdims=True) / float(m)


def _xconv1(pos3, prm, n, Cd, Cout, K, dil, dm, Tn):
    body = functools.partial(_xconv1_body, n, Tn, Cd, K, dil, dm)
    in_specs = [pl.BlockSpec((1, n, 3), lambda b, i: (b, 0, 0))]
    in_specs += [_bcast_spec(a.shape) for a in prm]
    return pl.pallas_call(
        body,
        grid=(_B, n // Tn),
        in_specs=in_specs,
        out_specs=(pl.BlockSpec((1, Tn, Cout), lambda b, i: (b, i, 0)),
                   pl.BlockSpec((1, 3, Tn), lambda b, i: (b, 0, i))),
        out_shape=(jax.ShapeDtypeStruct((_B, n, Cout), jnp.float32),
                   jax.ShapeDtypeStruct((_B, 3, n), jnp.float32)),
    )(pos3, *prm)


def _xconv2(idx3, pos3, x, prm, n, m, Cin, Cd, Cout, K, dil, dm):
    body = functools.partial(_xconv2_body, n, m, Cin, Cd, K, dil, dm)
    in_specs = [pl.BlockSpec((1, 1, m), lambda b: (b, 0, 0)),
                pl.BlockSpec((1, n, 3), lambda b: (b, 0, 0)),
                pl.BlockSpec((1, n, Cin), lambda b: (b, 0, 0))]
    in_specs += [_bcast_spec(a.shape) for a in prm]
    return pl.pallas_call(
        body,
        grid=(_B,),
        in_specs=in_specs,
        out_specs=(pl.BlockSpec((1, m, Cout), lambda b: (b, 0, 0)),
                   pl.BlockSpec((1, m, 3), lambda b: (b, 0, 0)),
                   pl.BlockSpec((1, 3, m), lambda b: (b, 0, 0))),
        out_shape=(jax.ShapeDtypeStruct((_B, m, Cout), jnp.float32),
                   jax.ShapeDtypeStruct((_B, m, 3), jnp.float32),
                   jax.ShapeDtypeStruct((_B, 3, m), jnp.float32)),
    )(idx3, pos3, x, *prm)


def _fps_idx_body(n, m, pos_ref, idx_ref):
    pos = pos_ref[...]
    iota_n = lax.broadcasted_iota(jnp.int32, (_B, n), 1).astype(jnp.float32)
    iota_m = lax.broadcasted_iota(jnp.int32, (1, m), 1)

    def body(i, carry):
        center, mind, idxmat = carry
        diff = pos - center[:, :, None]
        d = jnp.sum(diff * diff, axis=1)
        mind = jnp.minimum(mind, d)
        vals = jnp.max(mind, axis=1, keepdims=True)
        cand = jnp.where(mind == vals, iota_n, float(n))
        nxt = jnp.min(cand, axis=1, keepdims=True)
        sel = (iota_n == nxt).astype(jnp.float32)
        center = jnp.sum(pos * sel[:, None, :], axis=2)
        idxmat = idxmat + nxt.astype(jnp.int32) * (iota_m == i).astype(jnp.int32)
        return center, mind, idxmat

    center0 = pos[:, :, 0]
    mind0 = jnp.full((_B, n), 1e30, jnp.float32)
    idx0 = jnp.zeros((_B, m), jnp.int32)
    _, _, idxmat = lax.fori_loop(1, m, body, (center0, mind0, idx0))
    idx_ref[...] = idxmat


def _fps_idx(posT, m):
    n = posT.shape[2]
    idx = pl.pallas_call(
        functools.partial(_fps_idx_body, n, m),
        out_shape=jax.ShapeDtypeStruct((_B, m), jnp.int32),
    )(posT)
    return idx.reshape(_B, 1, m)


def _xconv34(idx3, pos3, x, prm3, prm4, n, m, Cin3, Cd3, Cd4, Cout4, K, dil):
    body = functools.partial(_xconv34_body, n, m, Cin3, Cd3, Cd4, K, dil)
    ins = [idx3, pos3, x] + list(prm3) + list(prm4)
    in_specs = [pl.BlockSpec((1, 1, m), lambda b: (b, 0, 0)),
                pl.BlockSpec((1, n, 3), lambda b: (b, 0, 0)),
                pl.BlockSpec((1, n, Cin3), lambda b: (b, 0, 0))]
    in_specs += [_bcast_spec(a.shape) for a in list(prm3) + list(prm4)]
    out = pl.pallas_call(
        body,
        grid=(_B,),
        in_specs=in_specs,
        out_specs=pl.BlockSpec((1, 1, Cout4), lambda b: (b, 0, 0)),
        out_shape=jax.ShapeDtypeStruct((_B, 1, Cout4), jnp.float32),
    )(*ins)
    return out.reshape(_B, Cout4)


def _head_body(x_ref, w1, b1, w2, b2, w3, b3, out_ref):
    x = x_ref[...]
    x = jnp.maximum(x @ w1[...] + b1[...], 0.0)
    x = jnp.maximum(x @ w2[...] + b2[...], 0.0)
    x = x @ w3[...] + b3[...]
    mx = jnp.max(x, axis=1, keepdims=True)
    sh = x - mx
    out_ref[...] = sh - jnp.log(jnp.sum(jnp.exp(sh), axis=1, keepdims=True))


def _head(xmean, lin1, lin2, lin3):
    nc = lin3['w'].shape[1]
    return pl.pallas_call(
        _head_body,
        out_shape=jax.ShapeDtypeStruct((_B, nc), jnp.float32),
    )(xmean, lin1['w'], _row(lin1['b']), lin2['w'], _row(lin2['b']),
      lin3['w'], _row(lin3['b']))


def kernel(pos, batch, params):
    del batch
    pos3 = pos.reshape(_B, _N0, 3)
    c1 = _prep_xconv_params(params['conv1'], 8, 2)
    c2 = _prep_xconv_params(params['conv2'], 12, 1)
    c3 = _prep_xconv_params(params['conv3'], 16, 1)
    c4 = _prep_xconv_params(params['conv4'], 16, 1)

    x1, posT = _xconv1(pos3, c1, _N0, 32, 48, 8, 1, 2, Tn=256)
    idx1 = _fps_idx(posT, 384)
    x2, pos2, pos2T = _xconv2(idx1, pos3, x1, c2, _N0, 384, 48, 64, 96,
                              12, 2, 1)
    idx2 = _fps_idx(pos2T, 129)
    xmean = _xconv34(idx2, pos2, x2, c3, c4, 384, 129, 96, 128, 256, 384,
                     16, 2)
    return _head(xmean, params['lin1'], params['lin2'], params['lin3'])
